# baseline (speedup 1.0000x reference)
import jax
import jax.numpy as jnp
from jax.experimental import pallas as pl
from jax.experimental.pallas import tpu as pltpu

_BLK = 128


def _gcn3_kernel(adj_ref, x_ref, w1_ref, b1_ref, w2_ref, b2_ref, w3_ref,
                 b3_ref, out_ref, s_ref, deg_ref, h1_ref):
    k = pl.program_id(0)
    nblk = pl.num_programs(0)
    blk = (adj_ref[...] != 0).astype(jnp.bfloat16)
    s_ref[pl.ds(k * _BLK, _BLK), :] = blk
    ones = jnp.ones((_BLK, 1), jnp.bfloat16)
    part = jax.lax.dot_general(
        blk, ones, (((0,), (0,)), ((), ())), preferred_element_type=jnp.float32)

    @pl.when(k == 0)
    def _():
        deg_ref[...] = 1.0 + part
        h1_ref[...] = jnp.dot(x_ref[...], w1_ref[...],
                              preferred_element_type=jnp.float32)

    @pl.when(k > 0)
    def _():
        deg_ref[...] += part

    @pl.when(k == nblk - 1)
    def _():
        dis = jax.lax.rsqrt(deg_ref[...])
        dis2 = dis * dis
        S = s_ref[...]
        h = h1_ref[...]
        x = None
        for w_ref, b_ref in ((None, b1_ref), (w2_ref, b2_ref),
                             (w3_ref, b3_ref)):
            if x is not None:
                h = jnp.dot(x, w_ref[...], preferred_element_type=jnp.float32)
            y = dis * h
            agg = jax.lax.dot_general(
                S, y, (((0,), (0,)), ((), ())),
                preferred_element_type=jnp.float32)
            x = jnp.maximum(dis * agg + dis2 * h + b_ref[...], 0.0)
        out_ref[...] = x


def kernel(bone_features, bone_adj, W1, b1, W2, b2, W3, b3):
    n, d = bone_features.shape
    d_out = W3.shape[1]
    full = lambda shape: pl.BlockSpec(shape, lambda k: (0, 0))
    return pl.pallas_call(
        _gcn3_kernel,
        grid=(n // _BLK,),
        in_specs=[
            pl.BlockSpec((_BLK, n), lambda k: (k, 0)),
            full((n, d)),
            full(W1.shape), full((1, W1.shape[1])),
            full(W2.shape), full((1, W2.shape[1])),
            full(W3.shape), full((1, W3.shape[1])),
        ],
        out_specs=full((n, d_out)),
        out_shape=jax.ShapeDtypeStruct((n, d_out), jnp.float32),
        scratch_shapes=[
            pltpu.VMEM((n, n), jnp.bfloat16),
            pltpu.VMEM((n, 1), jnp.float32),
            pltpu.VMEM((n, W1.shape[1]), jnp.float32),
        ],
    )(bone_adj, bone_features,
      W1, b1.reshape(1, -1), W2, b2.reshape(1, -1), W3, b3.reshape(1, -1))
